# baseline (speedup 1.0000x reference)
import jax
import jax.numpy as jnp
from jax import lax
from jax.experimental import pallas as pl
from jax.experimental.pallas import tpu as pltpu
from jax.experimental.pallas import tpu_sc as plsc

N = 262144
B = 8
G = 32
NSEG = B * G * G * G
C_PT = 128
H1 = 256
C_OUT = 128
EPS = 1e-5

_BN = 2048


def _pass_a_body(nocs_ref, xyz_ref, bidx_ref, ppf_ref, w1a_ref, w1b_ref, b1_ref,
                 flat_ref, extra_ref, s1_ref, ss1_ref, hist_ref):
    i = pl.program_id(0)
    nocs = nocs_ref[...]
    gs1 = jnp.float32(G - 1)
    idx_f = jnp.clip(jnp.round(nocs * gs1), 0.0, gs1)
    idx = idx_f.astype(jnp.int32)
    bidx = bidx_ref[...]
    flat = (bidx[:, 0] * (G * G * G)
            + idx[:, 0] * (G * G) + idx[:, 1] * G + idx[:, 2])
    flat_ref[...] = flat[:, None]
    grid_pts = idx_f * (1.0 / gs1)
    lo = nocs - grid_pts
    xyz = xyz_ref[...]
    zeros2 = jnp.zeros((lo.shape[0], 2), jnp.float32)
    extra = jnp.concatenate([lo, xyz, zeros2], axis=1)
    extra_ref[...] = extra
    p1 = (jnp.dot(ppf_ref[...], w1a_ref[...], preferred_element_type=jnp.float32)
          + jnp.dot(extra, w1b_ref[...], preferred_element_type=jnp.float32)
          + b1_ref[...])
    h = jnp.maximum(p1, 0.0)

    @pl.when(i == 0)
    def _():
        s1_ref[...] = jnp.zeros_like(s1_ref)
        ss1_ref[...] = jnp.zeros_like(ss1_ref)
        hist_ref[...] = jnp.zeros_like(hist_ref)

    s1_ref[...] += jnp.sum(h, axis=0, keepdims=True)
    ss1_ref[...] += jnp.sum(h * h, axis=0, keepdims=True)
    b8 = lax.broadcasted_iota(jnp.int32, (1, 8), 1)
    hist_ref[...] += jnp.sum((bidx == b8).astype(jnp.int32), axis=0,
                             keepdims=True)


def _pass_b_body(ppf_ref, extra_ref, w1a_ref, w1b_ref, b1_ref,
                 s1_ref, ss1_ref, g1_ref, bt1_ref, w2_ref, b2_ref,
                 h2_ref, s2_ref, ss2_ref):
    i = pl.program_id(0)
    mu1 = s1_ref[...] * (1.0 / N)
    var1 = ss1_ref[...] * (1.0 / N) - mu1 * mu1
    a1 = g1_ref[...] * lax.rsqrt(var1 + EPS)
    c1 = bt1_ref[...] - mu1 * a1
    p1 = (jnp.dot(ppf_ref[...], w1a_ref[...], preferred_element_type=jnp.float32)
          + jnp.dot(extra_ref[...], w1b_ref[...], preferred_element_type=jnp.float32)
          + b1_ref[...])
    h1 = jnp.maximum(p1, 0.0)
    h1s = h1 * a1
    p2 = (jnp.dot(h1s, w2_ref[...], preferred_element_type=jnp.float32)
          + jnp.dot(c1, w2_ref[...], preferred_element_type=jnp.float32)
          + b2_ref[...])
    h2 = jnp.maximum(p2, 0.0)
    h2_ref[...] = h2

    @pl.when(i == 0)
    def _():
        s2_ref[...] = jnp.zeros_like(s2_ref)
        ss2_ref[...] = jnp.zeros_like(ss2_ref)

    s2_ref[...] += jnp.sum(h2, axis=0, keepdims=True)
    ss2_ref[...] += jnp.sum(h2 * h2, axis=0, keepdims=True)


def _pass_d_body(sums_ref, cnt_ref, s2_ref, ss2_ref, g2_ref, bt2_ref, out_ref):
    mu2 = s2_ref[...] * (1.0 / N)
    var2 = ss2_ref[...] * (1.0 / N) - mu2 * mu2
    a2 = g2_ref[...] * lax.rsqrt(var2 + EPS)
    c2 = bt2_ref[...] - mu2 * a2
    cnt = cnt_ref[...]
    mean = sums_ref[...] / jnp.maximum(cnt, 1.0)
    o = mean * a2 + jnp.where(cnt > 0.0, c2, 0.0)
    out_ref[...] = jnp.transpose(o)[None]


_NTILE = 16
_NCORE = 2
_SEGCH = 4096
_CH_PER_CORE = (NSEG // _SEGCH) // _NCORE
_CH_PER_BATCH = (G * G * G) // _SEGCH
_SWEEP = 4096
_RB = 128
_GARB = _SEGCH
_TROW = _SEGCH // _NTILE
_SENT = 2 * _SWEEP


def _pass_c_body(flat_hbm, off_hbm, h2_hbm, za_hbm, sums_hbm, cnt_hbm,
                 idx_t, packed_t, pidb0, locb0, pidb1, locb1, rows0, rows1,
                 zbuf, cnt_t, cgath_t, cnt1d_t, off_t, acc_sh, acc2_sh,
                 cstage_sh, gsem0, gsem1, asem0, asem1):
    core = lax.axis_index("c")
    s = lax.axis_index("s")
    lane = lax.iota(jnp.int32, 16)
    zl = lane * 0
    fones = jnp.full((16,), 1.0, jnp.float32)
    pltpu.sync_copy(za_hbm, zbuf)
    pltpu.sync_copy(off_hbm, off_t)
    offv = off_t[...]
    padv = jnp.full((16,), jnp.int32(_GARB), jnp.int32) + s
    row0 = s * _TROW

    def chunk_body(c, _c):
        cg = core * _CH_PER_CORE + c
        seg_base = cg * _SEGCH
        bb = cg // _CH_PER_BATCH
        lo = jnp.sum(jnp.where(lane == bb, offv, 0))
        hi = jnp.sum(jnp.where(lane == bb + 1, offv, 0))
        lo16 = (lo // 16) * 16
        span = hi - lo16
        per = ((span + 255) // 256) * 16
        start = lo16 + s * per
        nsw = (per + (_SWEEP - 1)) // _SWEEP

        for q in range(_TROW // 64):
            pltpu.sync_copy(zbuf, acc_sh.at[pl.ds(row0 + q * 64, 64)])
            pltpu.sync_copy(zbuf, acc2_sh.at[pl.ds(row0 + q * 64, 64)])
        zv16 = jnp.zeros((16,), jnp.float32)

        def zrow(r, _r):
            for c8 in range(8):
                cnt_t[0, r, pl.ds(c8 * 16, 16)] = zv16
            return _r
        lax.fori_loop(0, _SEGCH // C_OUT, zrow, jnp.int32(0))
        plsc.subcore_barrier()

        def sweep_body(sw, tot):
            sbase = start + sw * _SWEEP
            pltpu.sync_copy(flat_hbm.at[pl.ds(sbase, _SWEEP)], idx_t)
            ng = jnp.minimum(per - sw * _SWEEP, _SWEEP) // 16

            def grp(g, w):
                v = idx_t[pl.ds(g * 16, 16)]
                loc = v - seg_base
                m = (loc >= 0) & (loc < _SEGCH)
                lr = jnp.clip(loc, 0, _SEGCH - 1)
                plsc.addupdate_scatter(cnt_t, [zl, lr >> 7, lr & 127],
                                       fones, mask=m)
                pid_u = (sbase + g * 16 + lane).astype(jnp.uint32)
                packed = (pid_u << 14) | (loc.astype(jnp.uint32)
                                          & jnp.uint32(16383))
                pre = plsc.cumsum(m.astype(jnp.int32))
                plsc.store_scatter(packed_t, [w + pre - 1],
                                   plsc.bitcast(packed, jnp.int32), mask=m)
                return w + jnp.sum(m.astype(jnp.int32))
            w = lax.fori_loop(0, ng, grp, jnp.int32(0))

            for k in range(_RB // 16):
                packed_t[pl.ds(w + k * 16, 16)] = padv
            nb = (w + (_RB - 1)) // _RB

            pidbs = (pidb0, pidb1)
            locbs = (locb0, locb1)
            rowsb = (rows0, rows1)
            accs = (acc_sh, acc2_sh)
            gsems = (gsem0, gsem1)
            asems = (asem0, asem1)
            NBMAX = _SWEEP // _RB + 1
            gdesc = [None] * NBMAX
            adesc = [None] * NBMAX
            for j in range(NBMAX + 1):
                if j >= 1:
                    jj = j - 1
                    @pl.when(jj < nb)
                    def _(jj=jj):
                        gdesc[jj].wait()
                        if jj >= 2:
                            adesc[jj - 2].wait()
                        adesc[jj] = pltpu.async_copy(
                            rowsb[jj % 2], accs[jj % 2].at[locbs[jj % 2]],
                            asems[jj % 2], add=True)
                if j < NBMAX:
                    @pl.when(j < nb)
                    def _(j=j):
                        t = j % 2
                        for k in range(_RB // 16):
                            vv = plsc.bitcast(
                                packed_t[pl.ds(j * _RB + k * 16, 16)],
                                jnp.uint32)
                            pidbs[t][pl.ds(k * 16, 16)] = jnp.minimum(
                                (vv >> jnp.uint32(14)).astype(jnp.int32),
                                N - 1)
                            locbs[t][pl.ds(k * 16, 16)] = jnp.minimum(
                                (vv & jnp.uint32(16383)).astype(jnp.int32),
                                _SEGCH + 15)
                        gdesc[j] = pltpu.async_copy(h2_hbm.at[pidbs[t]],
                                                    rowsb[t], gsems[t])
            for jj in range(NBMAX):
                @pl.when((jj + 2 > nb) & (jj < nb))
                def _(jj=jj):
                    adesc[jj].wait()
            return tot + nb
        lax.fori_loop(0, nsw, sweep_body, jnp.int32(0))
        pltpu.sync_copy(cnt_t.at[0], cstage_sh.at[s])
        plsc.subcore_barrier()

        for half in range(2):
            r0 = row0 + half * _RB
            pltpu.sync_copy(acc_sh.at[pl.ds(r0, _RB)], rows0)
            pltpu.sync_copy(acc2_sh.at[pl.ds(r0, _RB)], rows1)

            def addrow(r, _r):
                for c8 in range(8):
                    rows0[r, pl.ds(c8 * 16, 16)] = (
                        rows0[r, pl.ds(c8 * 16, 16)]
                        + rows1[r, pl.ds(c8 * 16, 16)])
                return _r
            lax.fori_loop(0, _RB, addrow, jnp.int32(0))
            pltpu.sync_copy(rows0, sums_hbm.at[pl.ds(seg_base + r0, _RB)])
        pltpu.sync_copy(cstage_sh.at[:, pl.ds(2 * s, 2)], cgath_t)
        for r in range(2):
            for c8 in range(8):
                acc16 = cgath_t[0, r, pl.ds(c8 * 16, 16)]
                for t in range(1, _NTILE):
                    acc16 = acc16 + cgath_t[t, r, pl.ds(c8 * 16, 16)]
                cnt1d_t[pl.ds(r * 128 + c8 * 16, 16)] = acc16
        pltpu.sync_copy(cnt1d_t, cnt_hbm.at[pl.ds(seg_base + row0, _TROW)])
        return _c
    lax.fori_loop(0, _CH_PER_CORE, chunk_body, jnp.int32(0))


def _segment_sums(flat_pad, off, h2):
    mesh = plsc.VectorSubcoreMesh(core_axis_name="c", subcore_axis_name="s")
    f = pl.kernel(
        _pass_c_body,
        out_type=[jax.ShapeDtypeStruct((NSEG, C_OUT), jnp.float32),
                  jax.ShapeDtypeStruct((NSEG,), jnp.float32)],
        mesh=mesh,
        compiler_params=pltpu.CompilerParams(needs_layout_passes=False),
        scratch_types=[
            pltpu.VMEM((_SWEEP,), jnp.int32),
            pltpu.VMEM((_SWEEP + _RB,), jnp.int32),
            pltpu.VMEM((_RB,), jnp.int32),
            pltpu.VMEM((_RB,), jnp.int32),
            pltpu.VMEM((_RB,), jnp.int32),
            pltpu.VMEM((_RB,), jnp.int32),
            pltpu.VMEM((_RB, C_OUT), jnp.float32),
            pltpu.VMEM((_RB, C_OUT), jnp.float32),
            pltpu.VMEM((64, C_OUT), jnp.float32),
            pltpu.VMEM((1, _SEGCH // C_OUT, C_OUT), jnp.float32),
            pltpu.VMEM((_NTILE, 2, C_OUT), jnp.float32),
            pltpu.VMEM((_TROW,), jnp.float32),
            pltpu.VMEM((16,), jnp.int32),
            pltpu.VMEM_SHARED((_SEGCH + 16, C_OUT), jnp.float32),
            pltpu.VMEM_SHARED((_SEGCH + 16, C_OUT), jnp.float32),
            pltpu.VMEM_SHARED((_NTILE, _SEGCH // C_OUT, C_OUT),
                              jnp.float32),
            pltpu.SemaphoreType.DMA,
            pltpu.SemaphoreType.DMA,
            pltpu.SemaphoreType.DMA,
            pltpu.SemaphoreType.DMA,
        ],
    )
    za = jnp.zeros((64, C_OUT), jnp.float32)
    sums, cnt = f(flat_pad, off, h2, za)
    return sums, cnt


def kernel(xyz, pred_nocs, per_point_features, pred_confidence, batch_size,
           batch_idx, W1, b1, g1, bt1, W2, b2, g2, bt2):
    del pred_confidence, batch_size
    bidx2 = batch_idx.astype(jnp.int32).reshape(N, 1)
    w1a = W1[:C_PT]
    w1b = jnp.concatenate([W1[C_PT:], jnp.zeros((2, H1), jnp.float32)], axis=0)
    b1r = b1.reshape(1, H1)
    g1r = g1.reshape(1, H1)
    bt1r = bt1.reshape(1, H1)
    b2r = b2.reshape(1, C_OUT)
    g2r = g2.reshape(1, C_OUT)
    bt2r = bt2.reshape(1, C_OUT)

    nsteps = N // _BN
    row_spec = lambda w: pl.BlockSpec((_BN, w), lambda i: (i, 0))
    full_spec = lambda a, b: pl.BlockSpec((a, b), lambda i: (0, 0))

    flat, extra, s1, ss1, hist = pl.pallas_call(
        _pass_a_body,
        grid=(nsteps,),
        in_specs=[row_spec(3), row_spec(3), row_spec(1), row_spec(C_PT),
                  full_spec(C_PT, H1), full_spec(8, H1), full_spec(1, H1)],
        out_specs=[row_spec(1), row_spec(8), full_spec(1, H1), full_spec(1, H1),
                   full_spec(1, 8)],
        out_shape=[jax.ShapeDtypeStruct((N, 1), jnp.int32),
                   jax.ShapeDtypeStruct((N, 8), jnp.float32),
                   jax.ShapeDtypeStruct((1, H1), jnp.float32),
                   jax.ShapeDtypeStruct((1, H1), jnp.float32),
                   jax.ShapeDtypeStruct((1, 8), jnp.int32)],
        compiler_params=pltpu.CompilerParams(
            dimension_semantics=("arbitrary",)),
    )(pred_nocs, xyz, bidx2, per_point_features, w1a, w1b, b1r)

    h2, s2, ss2 = pl.pallas_call(
        _pass_b_body,
        grid=(nsteps,),
        in_specs=[row_spec(C_PT), row_spec(8),
                  full_spec(C_PT, H1), full_spec(8, H1), full_spec(1, H1),
                  full_spec(1, H1), full_spec(1, H1), full_spec(1, H1),
                  full_spec(1, H1), full_spec(H1, C_OUT), full_spec(1, C_OUT)],
        out_specs=[row_spec(C_OUT), full_spec(1, C_OUT), full_spec(1, C_OUT)],
        out_shape=[jax.ShapeDtypeStruct((N, C_OUT), jnp.float32),
                   jax.ShapeDtypeStruct((1, C_OUT), jnp.float32),
                   jax.ShapeDtypeStruct((1, C_OUT), jnp.float32)],
        compiler_params=pltpu.CompilerParams(
            dimension_semantics=("arbitrary",)),
    )(per_point_features, extra, w1a, w1b, b1r, s1, ss1, g1r, bt1r, W2, b2r)

    off = jnp.concatenate([jnp.zeros((1,), jnp.int32),
                           jnp.cumsum(hist[0]).astype(jnp.int32),
                           jnp.full((7,), N, jnp.int32)])
    flat_pad = jnp.concatenate([flat[:, 0],
                                jnp.full((2 * 4096,), 1 << 20, jnp.int32)])
    sums, counts = _segment_sums(flat_pad, off, h2)
    counts = counts.reshape(NSEG, 1)

    bS = 2048
    dsteps = NSEG // bS
    per_b = (G * G * G) // bS
    out = pl.pallas_call(
        _pass_d_body,
        grid=(dsteps,),
        in_specs=[pl.BlockSpec((bS, C_OUT), lambda i: (i, 0)),
                  pl.BlockSpec((bS, 1), lambda i: (i, 0)),
                  full_spec(1, C_OUT), full_spec(1, C_OUT),
                  full_spec(1, C_OUT), full_spec(1, C_OUT)],
        out_specs=pl.BlockSpec((1, C_OUT, bS),
                               lambda i: (i // per_b, 0, i % per_b)),
        out_shape=jax.ShapeDtypeStruct((B, C_OUT, G * G * G), jnp.float32),
        compiler_params=pltpu.CompilerParams(
            dimension_semantics=("arbitrary",)),
    )(sums, counts, s2, ss2, g2r, bt2r)

    return out.reshape(B, C_OUT, G, G, G)


# speedup vs baseline: 2.4402x; 2.4402x over previous
import jax
import jax.numpy as jnp
from jax import lax
from jax.experimental import pallas as pl
from jax.experimental.pallas import tpu as pltpu
from jax.experimental.pallas import tpu_sc as plsc

N = 262144
B = 8
G = 32
NSEG = B * G * G * G
C_PT = 128
H1 = 256
C_OUT = 128
EPS = 1e-5

_BN = 2048


def _pass_a_body(nocs_ref, xyz_ref, bidx_ref, ppf_ref, w1a_ref, w1b_ref, b1_ref,
                 flat_ref, extra_ref, s1_ref, ss1_ref, hist_ref):
    i = pl.program_id(0)
    nocs = nocs_ref[...]
    gs1 = jnp.float32(G - 1)
    idx_f = jnp.clip(jnp.round(nocs * gs1), 0.0, gs1)
    idx = idx_f.astype(jnp.int32)
    bidx = bidx_ref[...]
    flat = (bidx[:, 0] * (G * G * G)
            + idx[:, 0] * (G * G) + idx[:, 1] * G + idx[:, 2])
    flat_ref[...] = flat[:, None]
    grid_pts = idx_f * (1.0 / gs1)
    lo = nocs - grid_pts
    xyz = xyz_ref[...]
    zeros2 = jnp.zeros((lo.shape[0], 2), jnp.float32)
    extra = jnp.concatenate([lo, xyz, zeros2], axis=1)
    extra_ref[...] = extra
    p1 = (jnp.dot(ppf_ref[...], w1a_ref[...], preferred_element_type=jnp.float32)
          + jnp.dot(extra, w1b_ref[...], preferred_element_type=jnp.float32)
          + b1_ref[...])
    h = jnp.maximum(p1, 0.0)

    @pl.when(i == 0)
    def _():
        s1_ref[...] = jnp.zeros_like(s1_ref)
        ss1_ref[...] = jnp.zeros_like(ss1_ref)
        hist_ref[...] = jnp.zeros_like(hist_ref)

    s1_ref[...] += jnp.sum(h, axis=0, keepdims=True)
    ss1_ref[...] += jnp.sum(h * h, axis=0, keepdims=True)
    b8 = lax.broadcasted_iota(jnp.int32, (1, 8), 1)
    hist_ref[...] += jnp.sum((bidx == b8).astype(jnp.int32), axis=0,
                             keepdims=True)


def _pass_b_body(ppf_ref, extra_ref, w1a_ref, w1b_ref, b1_ref,
                 s1_ref, ss1_ref, g1_ref, bt1_ref, w2_ref, b2_ref,
                 h2_ref, s2_ref, ss2_ref):
    i = pl.program_id(0)
    mu1 = s1_ref[...] * (1.0 / N)
    var1 = ss1_ref[...] * (1.0 / N) - mu1 * mu1
    a1 = g1_ref[...] * lax.rsqrt(var1 + EPS)
    c1 = bt1_ref[...] - mu1 * a1
    p1 = (jnp.dot(ppf_ref[...], w1a_ref[...], preferred_element_type=jnp.float32)
          + jnp.dot(extra_ref[...], w1b_ref[...], preferred_element_type=jnp.float32)
          + b1_ref[...])
    h1 = jnp.maximum(p1, 0.0)
    h1s = h1 * a1
    p2 = (jnp.dot(h1s, w2_ref[...], preferred_element_type=jnp.float32)
          + jnp.dot(c1, w2_ref[...], preferred_element_type=jnp.float32)
          + b2_ref[...])
    h2 = jnp.maximum(p2, 0.0)
    h2_ref[...] = h2

    @pl.when(i == 0)
    def _():
        s2_ref[...] = jnp.zeros_like(s2_ref)
        ss2_ref[...] = jnp.zeros_like(ss2_ref)

    s2_ref[...] += jnp.sum(h2, axis=0, keepdims=True)
    ss2_ref[...] += jnp.sum(h2 * h2, axis=0, keepdims=True)


def _pass_d_body(sums_ref, cnt_ref, s2_ref, ss2_ref, g2_ref, bt2_ref, out_ref):
    mu2 = s2_ref[...] * (1.0 / N)
    var2 = ss2_ref[...] * (1.0 / N) - mu2 * mu2
    a2 = g2_ref[...] * lax.rsqrt(var2 + EPS)
    c2 = bt2_ref[...] - mu2 * a2
    cnt = cnt_ref[...]
    mean = sums_ref[...] / jnp.maximum(cnt, 1.0)
    o = mean * a2 + jnp.where(cnt > 0.0, c2, 0.0)
    out_ref[...] = jnp.transpose(o)[None]


_NTILE = 16
_NCORE = 2
_SEGCH = 8192
_CH_PER_CORE = (NSEG // _SEGCH) // _NCORE
_CH_PER_BATCH = (G * G * G) // _SEGCH
_SWEEP = 2048
_RB = 128
_GARB = _SEGCH
_TROW = _SEGCH // _NTILE
_SENT = 8192


def _pass_c_body(flat_hbm, off_hbm, h2r_hbm, za_hbm, sums_hbm, cnt_hbm,
                 idx_t, locb0, locb1, rows0, rows1,
                 zbuf, cnt_t, cgath_t, cnt1d_t, off_t, acc_sh,
                 cstage_sh, gsem0, gsem1, asem0, asem1):
    core = lax.axis_index("c")
    s = lax.axis_index("s")
    lane = lax.iota(jnp.int32, 16)
    zl = lane * 0
    fones = jnp.full((16,), 1.0, jnp.float32)
    pltpu.sync_copy(za_hbm, zbuf)
    pltpu.sync_copy(off_hbm, off_t)
    offv = off_t[...]
    garb = jnp.full((16,), jnp.int32(_GARB), jnp.int32) + s
    row0 = s * _TROW
    locbs = (locb0, locb1)
    rowsb = (rows0, rows1)
    gsems = (gsem0, gsem1)
    asems = (asem0, asem1)

    def chunk_body(c, _c):
        cg = core * _CH_PER_CORE + c
        seg_base = cg * _SEGCH
        bb = cg // _CH_PER_BATCH
        lo = jnp.sum(jnp.where(lane == bb, offv, 0))
        hi = jnp.sum(jnp.where(lane == bb + 1, offv, 0))
        loB = lo // _RB
        span = hi - loB * _RB
        perB = (span + 16 * _RB - 1) // (16 * _RB)
        startB = loB + s * perB
        nsw = (perB * _RB + (_SWEEP - 1)) // _SWEEP

        for q in range(_TROW // 16):
            pltpu.sync_copy(zbuf, acc_sh.at[pl.ds(row0 + q * 16, 16)])
        zv16 = jnp.zeros((16,), jnp.float32)

        def zrow(r, _r):
            for c8 in range(8):
                cnt_t[0, r, pl.ds(c8 * 16, 16)] = zv16
            return _r
        lax.fori_loop(0, _SEGCH // C_OUT, zrow, jnp.int32(0))
        plsc.subcore_barrier()

        def sweep_body(sw, _s):
            swB = startB + sw * (_SWEEP // _RB)
            pltpu.sync_copy(flat_hbm.at[pl.ds(swB * _RB, _SWEEP)], idx_t)
            nb = jnp.minimum(perB - sw * (_SWEEP // _RB), _SWEEP // _RB)

            NBMAX = _SWEEP // _RB
            gdesc = [None] * NBMAX
            adesc = [None] * NBMAX
            srcs = [h2r_hbm.at[pl.ds(swB + j, 1)] for j in range(NBMAX)]
            for j in range(NBMAX + 1):
                if j >= 1:
                    jj = j - 1
                    @pl.when(jj < nb)
                    def _(jj=jj):
                        gdesc[jj].wait()
                        if jj >= 1:
                            adesc[jj - 1].wait()
                        adesc[jj] = pltpu.async_copy(
                            rowsb[jj % 2].at[0],
                            acc_sh.at[locbs[jj % 2]],
                            asems[jj % 2], add=True)
                if j < NBMAX:
                    @pl.when(j < nb)
                    def _(j=j):
                        t = j % 2
                        for k in range(_RB // 16):
                            v = idx_t[pl.ds(j * _RB + k * 16, 16)]
                            loc = v - seg_base
                            m = (loc >= 0) & (loc < _SEGCH)
                            lr = jnp.clip(loc, 0, _SEGCH - 1)
                            plsc.addupdate_scatter(
                                cnt_t, [zl, lr >> 7, lr & 127], fones, mask=m)
                            locbs[t][pl.ds(k * 16, 16)] = jnp.where(
                                m, loc, garb)
                        gdesc[j] = pltpu.async_copy(
                            srcs[j], rowsb[t], gsems[t])
            for jj in range(NBMAX):
                @pl.when(jj + 1 == nb)
                def _(jj=jj):
                    adesc[jj].wait()
            return _s
        lax.fori_loop(0, nsw, sweep_body, jnp.int32(0))
        pltpu.sync_copy(cnt_t.at[0], cstage_sh.at[s])
        plsc.subcore_barrier()

        pltpu.sync_copy(acc_sh.at[pl.ds(row0, _TROW)],
                        sums_hbm.at[pl.ds(seg_base + row0, _TROW)])
        for quar in range(4):
            pltpu.sync_copy(cstage_sh.at[:, pl.ds(4 * s + quar, 1)],
                            cgath_t)
            for c8 in range(8):
                acc16 = cgath_t[0, 0, pl.ds(c8 * 16, 16)]
                for t in range(1, _NTILE):
                    acc16 = acc16 + cgath_t[t, 0, pl.ds(c8 * 16, 16)]
                cnt1d_t[pl.ds(quar * 128 + c8 * 16, 16)] = acc16
        pltpu.sync_copy(cnt1d_t, cnt_hbm.at[pl.ds(seg_base + row0, _TROW)])
        return _c
    lax.fori_loop(0, _CH_PER_CORE, chunk_body, jnp.int32(0))


def _segment_sums(flat_pad, off, h2p):
    nblk = (N + _SENT) // _RB
    h2r = h2p.reshape(nblk, _RB, C_OUT)
    mesh = plsc.VectorSubcoreMesh(core_axis_name="c", subcore_axis_name="s")
    f = pl.kernel(
        _pass_c_body,
        out_type=[jax.ShapeDtypeStruct((NSEG, C_OUT), jnp.float32),
                  jax.ShapeDtypeStruct((NSEG,), jnp.float32)],
        mesh=mesh,
        compiler_params=pltpu.CompilerParams(needs_layout_passes=False),
        scratch_types=[
            pltpu.VMEM((_SWEEP,), jnp.int32),
            pltpu.VMEM((_RB,), jnp.int32),
            pltpu.VMEM((_RB,), jnp.int32),
            pltpu.VMEM((1, _RB, C_OUT), jnp.float32),
            pltpu.VMEM((1, _RB, C_OUT), jnp.float32),
            pltpu.VMEM((16, C_OUT), jnp.float32),
            pltpu.VMEM((1, _SEGCH // C_OUT, C_OUT), jnp.float32),
            pltpu.VMEM((_NTILE, 1, C_OUT), jnp.float32),
            pltpu.VMEM((_TROW,), jnp.float32),
            pltpu.VMEM((16,), jnp.int32),
            pltpu.VMEM_SHARED((_SEGCH + 16, C_OUT), jnp.float32),
            pltpu.VMEM_SHARED((_NTILE, _SEGCH // C_OUT, C_OUT),
                              jnp.float32),
            pltpu.SemaphoreType.DMA,
            pltpu.SemaphoreType.DMA,
            pltpu.SemaphoreType.DMA,
            pltpu.SemaphoreType.DMA,
        ],
    )
    za = jnp.zeros((16, C_OUT), jnp.float32)
    sums, cnt = f(flat_pad, off, h2r, za)
    return sums, cnt


def kernel(xyz, pred_nocs, per_point_features, pred_confidence, batch_size,
           batch_idx, W1, b1, g1, bt1, W2, b2, g2, bt2):
    del pred_confidence, batch_size
    bidx2 = batch_idx.astype(jnp.int32).reshape(N, 1)
    w1a = W1[:C_PT]
    w1b = jnp.concatenate([W1[C_PT:], jnp.zeros((2, H1), jnp.float32)], axis=0)
    b1r = b1.reshape(1, H1)
    g1r = g1.reshape(1, H1)
    bt1r = bt1.reshape(1, H1)
    b2r = b2.reshape(1, C_OUT)
    g2r = g2.reshape(1, C_OUT)
    bt2r = bt2.reshape(1, C_OUT)

    nsteps = N // _BN
    row_spec = lambda w: pl.BlockSpec((_BN, w), lambda i: (i, 0))
    full_spec = lambda a, b: pl.BlockSpec((a, b), lambda i: (0, 0))

    flat, extra, s1, ss1, hist = pl.pallas_call(
        _pass_a_body,
        grid=(nsteps,),
        in_specs=[row_spec(3), row_spec(3), row_spec(1), row_spec(C_PT),
                  full_spec(C_PT, H1), full_spec(8, H1), full_spec(1, H1)],
        out_specs=[row_spec(1), row_spec(8), full_spec(1, H1), full_spec(1, H1),
                   full_spec(1, 8)],
        out_shape=[jax.ShapeDtypeStruct((N, 1), jnp.int32),
                   jax.ShapeDtypeStruct((N, 8), jnp.float32),
                   jax.ShapeDtypeStruct((1, H1), jnp.float32),
                   jax.ShapeDtypeStruct((1, H1), jnp.float32),
                   jax.ShapeDtypeStruct((1, 8), jnp.int32)],
        compiler_params=pltpu.CompilerParams(
            dimension_semantics=("arbitrary",)),
    )(pred_nocs, xyz, bidx2, per_point_features, w1a, w1b, b1r)

    h2, s2, ss2 = pl.pallas_call(
        _pass_b_body,
        grid=(nsteps,),
        in_specs=[row_spec(C_PT), row_spec(8),
                  full_spec(C_PT, H1), full_spec(8, H1), full_spec(1, H1),
                  full_spec(1, H1), full_spec(1, H1), full_spec(1, H1),
                  full_spec(1, H1), full_spec(H1, C_OUT), full_spec(1, C_OUT)],
        out_specs=[row_spec(C_OUT), full_spec(1, C_OUT), full_spec(1, C_OUT)],
        out_shape=[jax.ShapeDtypeStruct((N + _SENT, C_OUT), jnp.float32),
                   jax.ShapeDtypeStruct((1, C_OUT), jnp.float32),
                   jax.ShapeDtypeStruct((1, C_OUT), jnp.float32)],
        compiler_params=pltpu.CompilerParams(
            dimension_semantics=("arbitrary",)),
    )(per_point_features, extra, w1a, w1b, b1r, s1, ss1, g1r, bt1r, W2, b2r)

    off = jnp.concatenate([jnp.zeros((1,), jnp.int32),
                           jnp.cumsum(hist[0]).astype(jnp.int32),
                           jnp.full((7,), N, jnp.int32)])
    flat_pad = jnp.concatenate([flat[:, 0],
                                jnp.full((_SENT,), 1 << 20, jnp.int32)])
    sums, counts = _segment_sums(flat_pad, off, h2)
    counts = counts.reshape(NSEG, 1)

    bS = 2048
    dsteps = NSEG // bS
    per_b = (G * G * G) // bS
    out = pl.pallas_call(
        _pass_d_body,
        grid=(dsteps,),
        in_specs=[pl.BlockSpec((bS, C_OUT), lambda i: (i, 0)),
                  pl.BlockSpec((bS, 1), lambda i: (i, 0)),
                  full_spec(1, C_OUT), full_spec(1, C_OUT),
                  full_spec(1, C_OUT), full_spec(1, C_OUT)],
        out_specs=pl.BlockSpec((1, C_OUT, bS),
                               lambda i: (i // per_b, 0, i % per_b)),
        out_shape=jax.ShapeDtypeStruct((B, C_OUT, G * G * G), jnp.float32),
        compiler_params=pltpu.CompilerParams(
            dimension_semantics=("arbitrary",)),
    )(sums, counts, s2, ss2, g2r, bt2r)

    return out.reshape(B, C_OUT, G, G, G)


# speedup vs baseline: 2.6371x; 1.0807x over previous
import jax
import jax.numpy as jnp
from jax import lax
from jax.experimental import pallas as pl
from jax.experimental.pallas import tpu as pltpu
from jax.experimental.pallas import tpu_sc as plsc

N = 262144
B = 8
G = 32
NSEG = B * G * G * G
C_PT = 128
H1 = 256
C_OUT = 128
EPS = 1e-5

_BN = 4096


def _pass_a_body(nocs_ref, xyz_ref, bidx_ref, ppf_ref, w1a_ref, w1b_ref, b1_ref,
                 flat_ref, extra_ref, s1_ref, ss1_ref, hist_ref):
    i = pl.program_id(0)
    nocs = nocs_ref[...]
    gs1 = jnp.float32(G - 1)
    idx_f = jnp.clip(jnp.round(nocs * gs1), 0.0, gs1)
    idx = idx_f.astype(jnp.int32)
    bidx = bidx_ref[...]
    flat = (bidx[:, 0] * (G * G * G)
            + idx[:, 0] * (G * G) + idx[:, 1] * G + idx[:, 2])
    flat_ref[...] = flat[:, None]
    grid_pts = idx_f * (1.0 / gs1)
    lo = nocs - grid_pts
    xyz = xyz_ref[...]
    zeros2 = jnp.zeros((lo.shape[0], 2), jnp.float32)
    extra = jnp.concatenate([lo, xyz, zeros2], axis=1)
    extra_ref[...] = extra
    p1 = (jnp.dot(ppf_ref[...], w1a_ref[...], preferred_element_type=jnp.float32)
          + jnp.dot(extra, w1b_ref[...], preferred_element_type=jnp.float32)
          + b1_ref[...])
    h = jnp.maximum(p1, 0.0)

    @pl.when(i == 0)
    def _():
        s1_ref[...] = jnp.zeros_like(s1_ref)
        ss1_ref[...] = jnp.zeros_like(ss1_ref)
        hist_ref[...] = jnp.zeros_like(hist_ref)

    s1_ref[...] += jnp.sum(h, axis=0, keepdims=True)
    ss1_ref[...] += jnp.sum(h * h, axis=0, keepdims=True)
    b8 = lax.broadcasted_iota(jnp.int32, (1, 8), 1)
    hist_ref[...] += jnp.sum((bidx == b8).astype(jnp.int32), axis=0,
                             keepdims=True)


def _pass_b_body(ppf_ref, extra_ref, w1a_ref, w1b_ref, b1_ref,
                 s1_ref, ss1_ref, g1_ref, bt1_ref, w2_ref, b2_ref,
                 h2_ref, s2_ref, ss2_ref):
    i = pl.program_id(0)
    mu1 = s1_ref[...] * (1.0 / N)
    var1 = ss1_ref[...] * (1.0 / N) - mu1 * mu1
    a1 = g1_ref[...] * lax.rsqrt(var1 + EPS)
    c1 = bt1_ref[...] - mu1 * a1
    p1 = (jnp.dot(ppf_ref[...], w1a_ref[...], preferred_element_type=jnp.float32)
          + jnp.dot(extra_ref[...], w1b_ref[...], preferred_element_type=jnp.float32)
          + b1_ref[...])
    h1 = jnp.maximum(p1, 0.0)
    h1s = h1 * a1
    p2 = (jnp.dot(h1s, w2_ref[...], preferred_element_type=jnp.float32)
          + jnp.dot(c1, w2_ref[...], preferred_element_type=jnp.float32)
          + b2_ref[...])
    h2 = jnp.maximum(p2, 0.0)
    h2_ref[...] = h2

    @pl.when(i == 0)
    def _():
        s2_ref[...] = jnp.zeros_like(s2_ref)
        ss2_ref[...] = jnp.zeros_like(ss2_ref)

    s2_ref[...] += jnp.sum(h2, axis=0, keepdims=True)
    ss2_ref[...] += jnp.sum(h2 * h2, axis=0, keepdims=True)


def _pass_d_body(sums_ref, cnt_ref, s2_ref, ss2_ref, g2_ref, bt2_ref, out_ref):
    mu2 = s2_ref[...] * (1.0 / N)
    var2 = ss2_ref[...] * (1.0 / N) - mu2 * mu2
    a2 = g2_ref[...] * lax.rsqrt(var2 + EPS)
    c2 = bt2_ref[...] - mu2 * a2
    cnt = cnt_ref[...]
    mean = sums_ref[...] / jnp.maximum(cnt, 1.0)
    o = mean * a2 + jnp.where(cnt > 0.0, c2, 0.0)
    out_ref[...] = jnp.transpose(o)[None]


_NTILE = 16
_NCORE = 2
_SEGCH = 8192
_CH_PER_CORE = (NSEG // _SEGCH) // _NCORE
_CH_PER_BATCH = (G * G * G) // _SEGCH
_SWEEP = 2048
_RB = 128
_GARB = _SEGCH
_TROW = _SEGCH // _NTILE
_SENT = 8192


def _pass_c_body(flat_hbm, off_hbm, h2r_hbm, za_hbm, sums_hbm, cnt_hbm,
                 idx_t, locb0, locb1, rows0, rows1,
                 zbuf, cnt_t, cgath_t, cnt1d_t, off_t, acc_sh,
                 cstage_sh, gsem0, gsem1, asem0, asem1):
    core = lax.axis_index("c")
    s = lax.axis_index("s")
    lane = lax.iota(jnp.int32, 16)
    zl = lane * 0
    fones = jnp.full((16,), 1.0, jnp.float32)
    pltpu.sync_copy(za_hbm, zbuf)
    pltpu.sync_copy(off_hbm, off_t)
    offv = off_t[...]
    garb = jnp.full((16,), jnp.int32(_GARB), jnp.int32) + s
    row0 = s * _TROW
    locbs = (locb0, locb1)
    rowsb = (rows0, rows1)
    gsems = (gsem0, gsem1)
    asems = (asem0, asem1)

    def chunk_body(c, _c):
        cg = core * _CH_PER_CORE + c
        seg_base = cg * _SEGCH
        bb = cg // _CH_PER_BATCH
        lo = jnp.sum(jnp.where(lane == bb, offv, 0))
        hi = jnp.sum(jnp.where(lane == bb + 1, offv, 0))
        loB = lo // _RB
        span = hi - loB * _RB
        perB = (span + 16 * _RB - 1) // (16 * _RB)
        startB = loB + s * perB
        nsw = (perB * _RB + (_SWEEP - 1)) // _SWEEP

        for q in range(_TROW // 16):
            pltpu.sync_copy(zbuf, acc_sh.at[pl.ds(row0 + q * 16, 16)])
        zv16 = jnp.zeros((16,), jnp.float32)

        def zrow(r, _r):
            for c8 in range(8):
                cnt_t[0, r, pl.ds(c8 * 16, 16)] = zv16
            return _r
        lax.fori_loop(0, _SEGCH // C_OUT, zrow, jnp.int32(0))
        plsc.subcore_barrier()

        def sweep_body(sw, _s):
            swB = startB + sw * (_SWEEP // _RB)
            pltpu.sync_copy(flat_hbm.at[pl.ds(swB * _RB, _SWEEP)], idx_t)
            nb = jnp.minimum(perB - sw * (_SWEEP // _RB), _SWEEP // _RB)

            NBMAX = _SWEEP // _RB
            gdesc = [None] * NBMAX
            adesc = [None] * NBMAX
            srcs = [h2r_hbm.at[pl.ds(swB + j, 1)] for j in range(NBMAX)]
            for j in range(NBMAX + 1):
                if j >= 1:
                    jj = j - 1
                    @pl.when(jj < nb)
                    def _(jj=jj):
                        gdesc[jj].wait()
                        if jj >= 1:
                            adesc[jj - 1].wait()
                        adesc[jj] = pltpu.async_copy(
                            rowsb[jj % 2].at[0],
                            acc_sh.at[locbs[jj % 2]],
                            asems[jj % 2], add=True)
                if j < NBMAX:
                    @pl.when(j < nb)
                    def _(j=j):
                        t = j % 2
                        for k in range(_RB // 16):
                            v = idx_t[pl.ds(j * _RB + k * 16, 16)]
                            loc = v - seg_base
                            m = (loc >= 0) & (loc < _SEGCH)
                            lr = jnp.clip(loc, 0, _SEGCH - 1)
                            plsc.addupdate_scatter(
                                cnt_t, [zl, lr >> 7, lr & 127], fones, mask=m)
                            locbs[t][pl.ds(k * 16, 16)] = jnp.where(
                                m, loc, garb)
                        gdesc[j] = pltpu.async_copy(
                            srcs[j], rowsb[t], gsems[t])
            for jj in range(NBMAX):
                @pl.when(jj + 1 == nb)
                def _(jj=jj):
                    adesc[jj].wait()
            return _s
        lax.fori_loop(0, nsw, sweep_body, jnp.int32(0))
        pltpu.sync_copy(cnt_t.at[0], cstage_sh.at[s])
        plsc.subcore_barrier()

        pltpu.sync_copy(acc_sh.at[pl.ds(row0, _TROW)],
                        sums_hbm.at[pl.ds(seg_base + row0, _TROW)])
        for quar in range(4):
            pltpu.sync_copy(cstage_sh.at[:, pl.ds(4 * s + quar, 1)],
                            cgath_t)
            for c8 in range(8):
                acc16 = cgath_t[0, 0, pl.ds(c8 * 16, 16)]
                for t in range(1, _NTILE):
                    acc16 = acc16 + cgath_t[t, 0, pl.ds(c8 * 16, 16)]
                cnt1d_t[pl.ds(quar * 128 + c8 * 16, 16)] = acc16
        pltpu.sync_copy(cnt1d_t, cnt_hbm.at[pl.ds(seg_base + row0, _TROW)])
        return _c
    lax.fori_loop(0, _CH_PER_CORE, chunk_body, jnp.int32(0))


def _segment_sums(flat_pad, off, h2p):
    nblk = (N + _SENT) // _RB
    h2r = h2p.reshape(nblk, _RB, C_OUT)
    mesh = plsc.VectorSubcoreMesh(core_axis_name="c", subcore_axis_name="s")
    f = pl.kernel(
        _pass_c_body,
        out_type=[jax.ShapeDtypeStruct((NSEG, C_OUT), jnp.float32),
                  jax.ShapeDtypeStruct((NSEG,), jnp.float32)],
        mesh=mesh,
        compiler_params=pltpu.CompilerParams(needs_layout_passes=False),
        scratch_types=[
            pltpu.VMEM((_SWEEP,), jnp.int32),
            pltpu.VMEM((_RB,), jnp.int32),
            pltpu.VMEM((_RB,), jnp.int32),
            pltpu.VMEM((1, _RB, C_OUT), jnp.float32),
            pltpu.VMEM((1, _RB, C_OUT), jnp.float32),
            pltpu.VMEM((16, C_OUT), jnp.float32),
            pltpu.VMEM((1, _SEGCH // C_OUT, C_OUT), jnp.float32),
            pltpu.VMEM((_NTILE, 1, C_OUT), jnp.float32),
            pltpu.VMEM((_TROW,), jnp.float32),
            pltpu.VMEM((16,), jnp.int32),
            pltpu.VMEM_SHARED((_SEGCH + 16, C_OUT), jnp.float32),
            pltpu.VMEM_SHARED((_NTILE, _SEGCH // C_OUT, C_OUT),
                              jnp.float32),
            pltpu.SemaphoreType.DMA,
            pltpu.SemaphoreType.DMA,
            pltpu.SemaphoreType.DMA,
            pltpu.SemaphoreType.DMA,
        ],
    )
    za = jnp.zeros((16, C_OUT), jnp.float32)
    sums, cnt = f(flat_pad, off, h2r, za)
    return sums, cnt


def kernel(xyz, pred_nocs, per_point_features, pred_confidence, batch_size,
           batch_idx, W1, b1, g1, bt1, W2, b2, g2, bt2):
    del pred_confidence, batch_size
    bidx2 = batch_idx.astype(jnp.int32).reshape(N, 1)
    w1a = W1[:C_PT]
    w1b = jnp.concatenate([W1[C_PT:], jnp.zeros((2, H1), jnp.float32)], axis=0)
    b1r = b1.reshape(1, H1)
    g1r = g1.reshape(1, H1)
    bt1r = bt1.reshape(1, H1)
    b2r = b2.reshape(1, C_OUT)
    g2r = g2.reshape(1, C_OUT)
    bt2r = bt2.reshape(1, C_OUT)

    nsteps = N // _BN
    row_spec = lambda w: pl.BlockSpec((_BN, w), lambda i: (i, 0))
    full_spec = lambda a, b: pl.BlockSpec((a, b), lambda i: (0, 0))

    flat, extra, s1, ss1, hist = pl.pallas_call(
        _pass_a_body,
        grid=(nsteps,),
        in_specs=[row_spec(3), row_spec(3), row_spec(1), row_spec(C_PT),
                  full_spec(C_PT, H1), full_spec(8, H1), full_spec(1, H1)],
        out_specs=[row_spec(1), row_spec(8), full_spec(1, H1), full_spec(1, H1),
                   full_spec(1, 8)],
        out_shape=[jax.ShapeDtypeStruct((N, 1), jnp.int32),
                   jax.ShapeDtypeStruct((N, 8), jnp.float32),
                   jax.ShapeDtypeStruct((1, H1), jnp.float32),
                   jax.ShapeDtypeStruct((1, H1), jnp.float32),
                   jax.ShapeDtypeStruct((1, 8), jnp.int32)],
        compiler_params=pltpu.CompilerParams(
            dimension_semantics=("arbitrary",)),
    )(pred_nocs, xyz, bidx2, per_point_features, w1a, w1b, b1r)

    h2, s2, ss2 = pl.pallas_call(
        _pass_b_body,
        grid=(nsteps,),
        in_specs=[row_spec(C_PT), row_spec(8),
                  full_spec(C_PT, H1), full_spec(8, H1), full_spec(1, H1),
                  full_spec(1, H1), full_spec(1, H1), full_spec(1, H1),
                  full_spec(1, H1), full_spec(H1, C_OUT), full_spec(1, C_OUT)],
        out_specs=[row_spec(C_OUT), full_spec(1, C_OUT), full_spec(1, C_OUT)],
        out_shape=[jax.ShapeDtypeStruct((N + _SENT, C_OUT), jnp.float32),
                   jax.ShapeDtypeStruct((1, C_OUT), jnp.float32),
                   jax.ShapeDtypeStruct((1, C_OUT), jnp.float32)],
        compiler_params=pltpu.CompilerParams(
            dimension_semantics=("arbitrary",)),
    )(per_point_features, extra, w1a, w1b, b1r, s1, ss1, g1r, bt1r, W2, b2r)

    off = jnp.concatenate([jnp.zeros((1,), jnp.int32),
                           jnp.cumsum(hist[0]).astype(jnp.int32),
                           jnp.full((7,), N, jnp.int32)])
    flat_pad = jnp.concatenate([flat[:, 0],
                                jnp.full((_SENT,), 1 << 20, jnp.int32)])
    sums, counts = _segment_sums(flat_pad, off, h2)
    counts = counts.reshape(NSEG, 1)

    bS = 4096
    dsteps = NSEG // bS
    per_b = (G * G * G) // bS
    out = pl.pallas_call(
        _pass_d_body,
        grid=(dsteps,),
        in_specs=[pl.BlockSpec((bS, C_OUT), lambda i: (i, 0)),
                  pl.BlockSpec((bS, 1), lambda i: (i, 0)),
                  full_spec(1, C_OUT), full_spec(1, C_OUT),
                  full_spec(1, C_OUT), full_spec(1, C_OUT)],
        out_specs=pl.BlockSpec((1, C_OUT, bS),
                               lambda i: (i // per_b, 0, i % per_b)),
        out_shape=jax.ShapeDtypeStruct((B, C_OUT, G * G * G), jnp.float32),
        compiler_params=pltpu.CompilerParams(
            dimension_semantics=("arbitrary",)),
    )(sums, counts, s2, ss2, g2r, bt2r)

    return out.reshape(B, C_OUT, G, G, G)


# speedup vs baseline: 2.6677x; 1.0116x over previous
import jax
import jax.numpy as jnp
from jax import lax
from jax.experimental import pallas as pl
from jax.experimental.pallas import tpu as pltpu
from jax.experimental.pallas import tpu_sc as plsc

N = 262144
B = 8
G = 32
NSEG = B * G * G * G
C_PT = 128
H1 = 256
C_OUT = 128
EPS = 1e-5

_BN = 4096


def _pass_a_body(nocs_ref, xyz_ref, bidx_ref, ppf_ref, w1a_ref, w1b_ref, b1_ref,
                 flat_ref, extra_ref, s1_ref, ss1_ref, hist_ref):
    i = pl.program_id(0)
    nocs = nocs_ref[...]
    gs1 = jnp.float32(G - 1)
    idx_f = jnp.clip(jnp.round(nocs * gs1), 0.0, gs1)
    idx = idx_f.astype(jnp.int32)
    bidx = bidx_ref[...]
    flat = (bidx[:, 0] * (G * G * G)
            + idx[:, 0] * (G * G) + idx[:, 1] * G + idx[:, 2])
    flat_ref[...] = flat[:, None]
    grid_pts = idx_f * (1.0 / gs1)
    lo = nocs - grid_pts
    xyz = xyz_ref[...]
    zeros2 = jnp.zeros((lo.shape[0], 2), jnp.float32)
    extra = jnp.concatenate([lo, xyz, zeros2], axis=1)
    extra_ref[...] = extra
    p1 = (jnp.dot(ppf_ref[...], w1a_ref[...], preferred_element_type=jnp.float32)
          + jnp.dot(extra, w1b_ref[...], preferred_element_type=jnp.float32)
          + b1_ref[...])
    h = jnp.maximum(p1, 0.0)

    @pl.when(i == 0)
    def _():
        s1_ref[...] = jnp.zeros_like(s1_ref)
        ss1_ref[...] = jnp.zeros_like(ss1_ref)
        hist_ref[...] = jnp.zeros_like(hist_ref)

    s1_ref[...] += jnp.sum(h, axis=0, keepdims=True)
    ss1_ref[...] += jnp.sum(h * h, axis=0, keepdims=True)
    b8 = lax.broadcasted_iota(jnp.int32, (1, 8), 1)
    hist_ref[...] += jnp.sum((bidx == b8).astype(jnp.int32), axis=0,
                             keepdims=True)


def _pass_b_body(ppf_ref, extra_ref, w1a_ref, w1b_ref, b1_ref,
                 s1_ref, ss1_ref, g1_ref, bt1_ref, w2_ref, b2_ref,
                 h2_ref, s2_ref, ss2_ref):
    i = pl.program_id(0)
    mu1 = s1_ref[...] * (1.0 / N)
    var1 = ss1_ref[...] * (1.0 / N) - mu1 * mu1
    a1 = g1_ref[...] * lax.rsqrt(var1 + EPS)
    c1 = bt1_ref[...] - mu1 * a1
    p1 = (jnp.dot(ppf_ref[...], w1a_ref[...], preferred_element_type=jnp.float32)
          + jnp.dot(extra_ref[...], w1b_ref[...], preferred_element_type=jnp.float32)
          + b1_ref[...])
    h1 = jnp.maximum(p1, 0.0)
    h1s = h1 * a1
    p2 = (jnp.dot(h1s, w2_ref[...], preferred_element_type=jnp.float32)
          + jnp.dot(c1, w2_ref[...], preferred_element_type=jnp.float32)
          + b2_ref[...])
    h2 = jnp.maximum(p2, 0.0)
    h2_ref[...] = h2

    @pl.when(i == 0)
    def _():
        s2_ref[...] = jnp.zeros_like(s2_ref)
        ss2_ref[...] = jnp.zeros_like(ss2_ref)

    s2_ref[...] += jnp.sum(h2, axis=0, keepdims=True)
    ss2_ref[...] += jnp.sum(h2 * h2, axis=0, keepdims=True)


def _pass_d_body(sums_ref, cnt_ref, s2_ref, ss2_ref, g2_ref, bt2_ref, out_ref):
    mu2 = s2_ref[...] * (1.0 / N)
    var2 = ss2_ref[...] * (1.0 / N) - mu2 * mu2
    a2 = g2_ref[...] * lax.rsqrt(var2 + EPS)
    c2 = bt2_ref[...] - mu2 * a2
    cnt = cnt_ref[...]
    mean = sums_ref[...] / jnp.maximum(cnt, 1.0)
    o = mean * a2 + jnp.where(cnt > 0.0, c2, 0.0)
    out_ref[...] = jnp.transpose(o)[None]


_NTILE = 16
_NCORE = 2
_SEGCH = 8192
_CH_PER_CORE = (NSEG // _SEGCH) // _NCORE
_CH_PER_BATCH = (G * G * G) // _SEGCH
_SWEEP = 2048
_RB = 128
_GARB = _SEGCH
_TROW = _SEGCH // _NTILE
_SENT = 8192


def _pass_c_body(flat_hbm, off_hbm, h2r_hbm, za_hbm, sums_hbm, cnt_hbm,
                 idx_t, locb0, locb1, rows0, rows1,
                 zbuf, cnt_t, cgath_t, cnt1d_t, off_t, acc_sh,
                 cstage_sh, gsem0, gsem1, asem0, asem1):
    core = lax.axis_index("c")
    s = lax.axis_index("s")
    lane = lax.iota(jnp.int32, 16)
    zl = lane * 0
    fones = jnp.full((16,), 1.0, jnp.float32)
    pltpu.sync_copy(za_hbm, zbuf)
    pltpu.sync_copy(off_hbm, off_t)
    offv = off_t[...]
    garb = jnp.full((16,), jnp.int32(_GARB), jnp.int32) + s
    row0 = s * _TROW
    locbs = (locb0, locb1)
    rowsb = (rows0, rows1)
    gsems = (gsem0, gsem1)
    asems = (asem0, asem1)

    def chunk_body(c, _c):
        cg = core * _CH_PER_CORE + c
        seg_base = cg * _SEGCH
        bb = cg // _CH_PER_BATCH
        lo = jnp.sum(jnp.where(lane == bb, offv, 0))
        hi = jnp.sum(jnp.where(lane == bb + 1, offv, 0))
        loB = lo // _RB
        span = hi - loB * _RB
        perB = (span + 16 * _RB - 1) // (16 * _RB)
        startB = loB + s * perB
        nsw = (perB * _RB + (_SWEEP - 1)) // _SWEEP

        for q in range(_TROW // 16):
            pltpu.sync_copy(zbuf, acc_sh.at[pl.ds(row0 + q * 16, 16)])
        zv16 = jnp.zeros((16,), jnp.float32)

        def zrow(r, _r):
            for c8 in range(8):
                cnt_t[0, r, pl.ds(c8 * 16, 16)] = zv16
            return _r
        lax.fori_loop(0, _SEGCH // C_OUT, zrow, jnp.int32(0))
        plsc.subcore_barrier()

        def sweep_body(sw, _s):
            swB = startB + sw * (_SWEEP // _RB)
            pltpu.sync_copy(flat_hbm.at[pl.ds(swB * _RB, _SWEEP)], idx_t)
            nb = jnp.minimum(perB - sw * (_SWEEP // _RB), _SWEEP // _RB)

            NBMAX = _SWEEP // _RB
            gdesc = [None] * NBMAX
            adesc = [None] * NBMAX
            srcs = [h2r_hbm.at[pl.ds(swB + j, 1)] for j in range(NBMAX)]
            for j in range(NBMAX + 1):
                if j >= 1:
                    jj = j - 1
                    @pl.when(jj < nb)
                    def _(jj=jj):
                        gdesc[jj].wait()
                        if jj >= 1:
                            adesc[jj - 1].wait()
                        adesc[jj] = pltpu.async_copy(
                            rowsb[jj % 2].at[0],
                            acc_sh.at[locbs[jj % 2]],
                            asems[jj % 2], add=True)
                if j < NBMAX:
                    @pl.when(j < nb)
                    def _(j=j):
                        t = j % 2
                        for k in range(_RB // 16):
                            v = idx_t[pl.ds(j * _RB + k * 16, 16)]
                            loc = v - seg_base
                            m = (loc >= 0) & (loc < _SEGCH)
                            lr = jnp.clip(loc, 0, _SEGCH - 1)
                            plsc.addupdate_scatter(
                                cnt_t, [zl, lr >> 7, lr & 127], fones, mask=m)
                            locbs[t][pl.ds(k * 16, 16)] = jnp.where(
                                m, loc, garb)
                        gdesc[j] = pltpu.async_copy(
                            srcs[j], rowsb[t], gsems[t])
            for jj in range(NBMAX):
                @pl.when(jj + 1 == nb)
                def _(jj=jj):
                    adesc[jj].wait()
            return _s
        lax.fori_loop(0, nsw, sweep_body, jnp.int32(0))
        pltpu.sync_copy(cnt_t.at[0], cstage_sh.at[s])
        plsc.subcore_barrier()

        pltpu.sync_copy(acc_sh.at[pl.ds(row0, _TROW)],
                        sums_hbm.at[pl.ds(seg_base + row0, _TROW)])
        for quar in range(4):
            pltpu.sync_copy(cstage_sh.at[:, pl.ds(4 * s + quar, 1)],
                            cgath_t)
            for c8 in range(8):
                acc16 = cgath_t[0, 0, pl.ds(c8 * 16, 16)]
                for t in range(1, _NTILE):
                    acc16 = acc16 + cgath_t[t, 0, pl.ds(c8 * 16, 16)]
                cnt1d_t[pl.ds(quar * 128 + c8 * 16, 16)] = acc16
        pltpu.sync_copy(cnt1d_t, cnt_hbm.at[pl.ds(seg_base + row0, _TROW)])
        return _c
    lax.fori_loop(0, _CH_PER_CORE, chunk_body, jnp.int32(0))


def _segment_sums(flat_pad, off, h2p):
    nblk = (N + _SENT) // _RB
    h2r = h2p.reshape(nblk, _RB, C_OUT)
    mesh = plsc.VectorSubcoreMesh(core_axis_name="c", subcore_axis_name="s")
    f = pl.kernel(
        _pass_c_body,
        out_type=[jax.ShapeDtypeStruct((NSEG, C_OUT), jnp.float32),
                  jax.ShapeDtypeStruct((NSEG,), jnp.float32)],
        mesh=mesh,
        compiler_params=pltpu.CompilerParams(needs_layout_passes=False),
        scratch_types=[
            pltpu.VMEM((_SWEEP,), jnp.int32),
            pltpu.VMEM((_RB,), jnp.int32),
            pltpu.VMEM((_RB,), jnp.int32),
            pltpu.VMEM((1, _RB, C_OUT), jnp.float32),
            pltpu.VMEM((1, _RB, C_OUT), jnp.float32),
            pltpu.VMEM((16, C_OUT), jnp.float32),
            pltpu.VMEM((1, _SEGCH // C_OUT, C_OUT), jnp.float32),
            pltpu.VMEM((_NTILE, 1, C_OUT), jnp.float32),
            pltpu.VMEM((_TROW,), jnp.float32),
            pltpu.VMEM((16,), jnp.int32),
            pltpu.VMEM_SHARED((_SEGCH + 16, C_OUT), jnp.float32),
            pltpu.VMEM_SHARED((_NTILE, _SEGCH // C_OUT, C_OUT),
                              jnp.float32),
            pltpu.SemaphoreType.DMA,
            pltpu.SemaphoreType.DMA,
            pltpu.SemaphoreType.DMA,
            pltpu.SemaphoreType.DMA,
        ],
    )
    za = jnp.zeros((16, C_OUT), jnp.float32)
    sums, cnt = f(flat_pad, off, h2r, za)
    return sums, cnt


def kernel(xyz, pred_nocs, per_point_features, pred_confidence, batch_size,
           batch_idx, W1, b1, g1, bt1, W2, b2, g2, bt2):
    del pred_confidence, batch_size
    bidx2 = batch_idx.astype(jnp.int32).reshape(N, 1)
    w1a = W1[:C_PT]
    w1b = jnp.concatenate([W1[C_PT:], jnp.zeros((2, H1), jnp.float32)], axis=0)
    b1r = b1.reshape(1, H1)
    g1r = g1.reshape(1, H1)
    bt1r = bt1.reshape(1, H1)
    b2r = b2.reshape(1, C_OUT)
    g2r = g2.reshape(1, C_OUT)
    bt2r = bt2.reshape(1, C_OUT)

    nsteps = N // _BN
    row_spec = lambda w: pl.BlockSpec((_BN, w), lambda i: (i, 0))
    full_spec = lambda a, b: pl.BlockSpec((a, b), lambda i: (0, 0))

    flat, extra, s1, ss1, hist = pl.pallas_call(
        _pass_a_body,
        grid=(nsteps,),
        in_specs=[row_spec(3), row_spec(3), row_spec(1), row_spec(C_PT),
                  full_spec(C_PT, H1), full_spec(8, H1), full_spec(1, H1)],
        out_specs=[row_spec(1), row_spec(8), full_spec(1, H1), full_spec(1, H1),
                   full_spec(1, 8)],
        out_shape=[jax.ShapeDtypeStruct((N, 1), jnp.int32),
                   jax.ShapeDtypeStruct((N, 8), jnp.float32),
                   jax.ShapeDtypeStruct((1, H1), jnp.float32),
                   jax.ShapeDtypeStruct((1, H1), jnp.float32),
                   jax.ShapeDtypeStruct((1, 8), jnp.int32)],
        compiler_params=pltpu.CompilerParams(
            dimension_semantics=("arbitrary",)),
    )(pred_nocs, xyz, bidx2, per_point_features, w1a, w1b, b1r)

    h2, s2, ss2 = pl.pallas_call(
        _pass_b_body,
        grid=(nsteps,),
        in_specs=[row_spec(C_PT), row_spec(8),
                  full_spec(C_PT, H1), full_spec(8, H1), full_spec(1, H1),
                  full_spec(1, H1), full_spec(1, H1), full_spec(1, H1),
                  full_spec(1, H1), full_spec(H1, C_OUT), full_spec(1, C_OUT)],
        out_specs=[row_spec(C_OUT), full_spec(1, C_OUT), full_spec(1, C_OUT)],
        out_shape=[jax.ShapeDtypeStruct((N + _SENT, C_OUT), jnp.float32),
                   jax.ShapeDtypeStruct((1, C_OUT), jnp.float32),
                   jax.ShapeDtypeStruct((1, C_OUT), jnp.float32)],
        compiler_params=pltpu.CompilerParams(
            dimension_semantics=("arbitrary",)),
    )(per_point_features, extra, w1a, w1b, b1r, s1, ss1, g1r, bt1r, W2, b2r)

    off = jnp.concatenate([jnp.zeros((1,), jnp.int32),
                           jnp.cumsum(hist[0]).astype(jnp.int32),
                           jnp.full((7,), N, jnp.int32)])
    flat_pad = jnp.concatenate([flat[:, 0],
                                jnp.full((_SENT,), 1 << 20, jnp.int32)])
    sums, counts = _segment_sums(flat_pad, off, h2)
    counts = counts.reshape(NSEG, 1)

    bS = 8192
    dsteps = NSEG // bS
    per_b = (G * G * G) // bS
    out = pl.pallas_call(
        _pass_d_body,
        grid=(dsteps,),
        in_specs=[pl.BlockSpec((bS, C_OUT), lambda i: (i, 0)),
                  pl.BlockSpec((bS, 1), lambda i: (i, 0)),
                  full_spec(1, C_OUT), full_spec(1, C_OUT),
                  full_spec(1, C_OUT), full_spec(1, C_OUT)],
        out_specs=pl.BlockSpec((1, C_OUT, bS),
                               lambda i: (i // per_b, 0, i % per_b)),
        out_shape=jax.ShapeDtypeStruct((B, C_OUT, G * G * G), jnp.float32),
        compiler_params=pltpu.CompilerParams(
            dimension_semantics=("arbitrary",)),
    )(sums, counts, s2, ss2, g2r, bt2r)

    return out.reshape(B, C_OUT, G, G, G)
